# baseline (speedup 1.0000x reference)
import functools

import jax
import jax.numpy as jnp
from jax import lax
from jax.experimental import pallas as pl
from jax.experimental.pallas import tpu as pltpu
from jax.experimental.pallas import tpu_sc as plsc

DIM = 128
NC = 2
NS = 16
NW = NC * NS
CHUNK = 128


@functools.partial(jax.jit, static_argnums=(3, 4, 5))
def _gather_pairs(pos3, cos_cached, sin_cached, B, S, n_chunks):
    mesh = plsc.VectorSubcoreMesh(
        core_axis_name="c", subcore_axis_name="s",
        num_cores=NC, num_subcores=NS)

    out_type = (
        jax.ShapeDtypeStruct((B, 1, S, DIM), jnp.float32),
        jax.ShapeDtypeStruct((B, 1, S, DIM), jnp.float32),
    )
    s_per_w = n_chunks * CHUNK

    NBUF = 3

    @functools.partial(
        pl.kernel,
        out_type=out_type,
        mesh=mesh,
        scratch_types=[
            pltpu.VMEM((n_chunks, CHUNK), jnp.int32),
            [pltpu.VMEM((CHUNK, DIM), jnp.float32) for _ in range(NBUF)],
            [pltpu.VMEM((CHUNK, DIM), jnp.float32) for _ in range(NBUF)],
            [pltpu.SemaphoreType.DMA for _ in range(4 * NBUF)],
        ],
    )
    def k(pos_hbm, cos_hbm, sin_hbm, cos_out, sin_out,
          idx_v, rows_c, rows_s, sems):
        gc, gs = sems[:NBUF], sems[NBUF:2 * NBUF]
        wc, ws = sems[2 * NBUF:3 * NBUF], sems[3 * NBUF:]
        wid = lax.axis_index("s") * NC + lax.axis_index("c")
        pltpu.sync_copy(pos_hbm.at[wid], idx_v)
        w_per_b = S // s_per_w
        b = wid // w_per_b
        s_base = (wid % w_per_b) * s_per_w

        def gather(ch):
            bf = ch % NBUF
            return (
                pltpu.async_copy(cos_hbm.at[idx_v.at[ch]], rows_c[bf], gc[bf]),
                pltpu.async_copy(sin_hbm.at[idx_v.at[ch]], rows_s[bf], gs[bf]),
            )

        g = {ch: gather(ch) for ch in range(min(NBUF, n_chunks))}
        w = {}
        for ch in range(n_chunks):
            bf = ch % NBUF
            out_slice = pl.ds(s_base + ch * CHUNK, CHUNK)
            g[ch][0].wait()
            w[ch] = [pltpu.async_copy(
                rows_c[bf], cos_out.at[b, 0, out_slice], wc[bf])]
            g[ch][1].wait()
            w[ch].append(pltpu.async_copy(
                rows_s[bf], sin_out.at[b, 0, out_slice], ws[bf]))
            nxt = ch + NBUF
            if nxt < n_chunks:
                for cpy in w[ch]:
                    cpy.wait()
                del w[ch]
                g[nxt] = gather(nxt)
        for ch in sorted(w):
            for cpy in w[ch]:
                cpy.wait()

    return k(pos3, cos_cached, sin_cached)


def kernel(x, position_ids, cos_cached, sin_cached):
    B, S = position_ids.shape
    n_chunks = (B * S) // (NW * CHUNK)
    pos3 = position_ids.astype(jnp.int32).reshape(NW, n_chunks, CHUNK)
    return _gather_pairs(pos3, cos_cached, sin_cached, B, S, n_chunks)


# speedup vs baseline: 1.0104x; 1.0104x over previous
import functools

import jax
import jax.numpy as jnp
from jax import lax
from jax.experimental import pallas as pl
from jax.experimental.pallas import tpu as pltpu
from jax.experimental.pallas import tpu_sc as plsc

DIM = 128
NC = 2
NS = 16
NW = NC * NS
CHUNK = 128


@functools.partial(jax.jit, static_argnums=(3, 4, 5))
def _gather_pairs(pos, cos_cached, sin_cached, B, S, n_chunks):
    mesh = plsc.VectorSubcoreMesh(
        core_axis_name="c", subcore_axis_name="s",
        num_cores=NC, num_subcores=NS)

    out_type = (
        jax.ShapeDtypeStruct((B, 1, S, DIM), jnp.float32),
        jax.ShapeDtypeStruct((B, 1, S, DIM), jnp.float32),
    )
    s_per_w = n_chunks * CHUNK

    NBUF = 3

    @functools.partial(
        pl.kernel,
        out_type=out_type,
        mesh=mesh,
        scratch_types=[
            pltpu.VMEM((n_chunks * CHUNK,), jnp.int32),
            [pltpu.VMEM((CHUNK, DIM), jnp.float32) for _ in range(NBUF)],
            [pltpu.VMEM((CHUNK, DIM), jnp.float32) for _ in range(NBUF)],
            [pltpu.SemaphoreType.DMA for _ in range(4 * NBUF)],
        ],
    )
    def k(pos_hbm, cos_hbm, sin_hbm, cos_out, sin_out,
          idx_v, rows_c, rows_s, sems):
        gc, gs = sems[:NBUF], sems[NBUF:2 * NBUF]
        wc, ws = sems[2 * NBUF:3 * NBUF], sems[3 * NBUF:]
        wid = lax.axis_index("s") * NC + lax.axis_index("c")
        w_per_b = S // s_per_w
        b = wid // w_per_b
        s_base = (wid % w_per_b) * s_per_w
        pltpu.sync_copy(pos_hbm.at[b, pl.ds(s_base, s_per_w)], idx_v)

        def gather(ch):
            bf = ch % NBUF
            idx = idx_v.at[pl.ds(ch * CHUNK, CHUNK)]
            return (
                pltpu.async_copy(cos_hbm.at[idx], rows_c[bf], gc[bf]),
                pltpu.async_copy(sin_hbm.at[idx], rows_s[bf], gs[bf]),
            )

        g = {ch: gather(ch) for ch in range(min(NBUF, n_chunks))}
        w = {}
        for ch in range(n_chunks):
            bf = ch % NBUF
            out_slice = pl.ds(s_base + ch * CHUNK, CHUNK)
            g[ch][0].wait()
            w[ch] = [pltpu.async_copy(
                rows_c[bf], cos_out.at[b, 0, out_slice], wc[bf])]
            g[ch][1].wait()
            w[ch].append(pltpu.async_copy(
                rows_s[bf], sin_out.at[b, 0, out_slice], ws[bf]))
            nxt = ch + NBUF
            if nxt < n_chunks:
                for cpy in w[ch]:
                    cpy.wait()
                del w[ch]
                g[nxt] = gather(nxt)
        for ch in sorted(w):
            for cpy in w[ch]:
                cpy.wait()

    return k(pos, cos_cached, sin_cached)


def kernel(x, position_ids, cos_cached, sin_cached):
    B, S = position_ids.shape
    n_chunks = (B * S) // (NW * CHUNK)
    pos = position_ids.astype(jnp.int32)
    return _gather_pairs(pos, cos_cached, sin_cached, B, S, n_chunks)
